# baseline (speedup 1.0000x reference)
import jax
import jax.numpy as jnp
from jax.experimental import pallas as pl
from jax.experimental.pallas import tpu as pltpu

HIDDEN = 256
NVOC = 5
NAUX = 60
BATCH_BLOCK = 16
NSPLIT = 4


def _body(ids_ref, aux_hbm, out_ref, buf, sems):
    g = BATCH_BLOCK
    i = pl.program_id(0)
    nb = pl.num_programs(0)
    s = ids_ref.shape[1]
    rows = s // NSPLIT

    def start(slot, blk):
        for c in range(NSPLIT):
            pltpu.make_async_copy(
                aux_hbm.at[pl.ds(blk * g, g), pl.ds(c * rows, rows)],
                buf.at[slot, :, pl.ds(c * rows, rows)],
                sems.at[slot, c],
            ).start()

    def wait(slot, blk):
        for c in range(NSPLIT):
            pltpu.make_async_copy(
                aux_hbm.at[pl.ds(blk * g, g), pl.ds(c * rows, rows)],
                buf.at[slot, :, pl.ds(c * rows, rows)],
                sems.at[slot, c],
            ).wait()

    @pl.when(i == 0)
    def _():
        start(0, 0)

    @pl.when(i + 1 < nb)
    def _():
        start((i + 1) % 2, i + 1)

    wait(i % 2, i)
    aux = buf[i % 2]
    ids = ids_ref[...][:, :, None]
    col = jax.lax.broadcasted_iota(jnp.int32, (g, s, HIDDEN), 2)
    oh = jnp.where((col == ids) & (col < NVOC), 1.0, 0.0).astype(jnp.float32)
    z_left = jnp.zeros((g, s, NVOC), jnp.float32)
    z_right = jnp.zeros((g, s, HIDDEN - NVOC - NAUX), jnp.float32)
    shifted = jnp.concatenate([z_left, aux, z_right], axis=-1)
    out_ref[...] = oh + shifted


def kernel(input_ids, aux_features):
    b, s = input_ids.shape
    g = BATCH_BLOCK
    return pl.pallas_call(
        _body,
        grid=(b // g,),
        in_specs=[
            pl.BlockSpec((g, s), lambda i: (i, 0)),
            pl.BlockSpec(memory_space=pltpu.MemorySpace.HBM),
        ],
        out_specs=pl.BlockSpec((g, s, HIDDEN), lambda i: (i, 0, 0)),
        out_shape=jax.ShapeDtypeStruct((b, s, HIDDEN), jnp.float32),
        scratch_shapes=[
            pltpu.VMEM((2, g, s, NAUX), jnp.float32),
            pltpu.SemaphoreType.DMA((2, NSPLIT)),
        ],
    )(input_ids, aux_features)
